# Initial kernel scaffold; baseline (speedup 1.0000x reference)
import jax
import jax.numpy as jnp
from jax.experimental import pallas as pl


def kernel(x0, x1, x2, W0, b0, W1, b1, W2, b2):
    raise NotImplementedError("write your pallas kernel here")


# speedup vs baseline: 1.2280x; 1.2280x over previous
import functools

import jax
import jax.numpy as jnp
from jax.experimental import pallas as pl

NA = 3
NO = 85


def _head_kernel(x_ref, wt_ref, b_ref, out_ref):
    z = jax.lax.dot_general(
        x_ref[0], wt_ref[...],
        dimension_numbers=(((0,), (0,)), ((), ())),
        preferred_element_type=jnp.float32,
    )
    z = z + b_ref[0]
    for a in range(NA):
        out_ref[0, a] = z[:, a * NO:(a + 1) * NO]


@functools.partial(jax.jit, static_argnames=("tile",))
def _head(x, W, b, tile):
    bs, c, ny, nx = x.shape
    hw = ny * nx
    xr = x.reshape(bs, c, hw)
    wt = W.T
    br = b.reshape(1, NA * NO)
    grid = (bs, hw // tile)
    out = pl.pallas_call(
        _head_kernel,
        grid=grid,
        in_specs=[
            pl.BlockSpec((1, c, tile), lambda i, j: (i, 0, j)),
            pl.BlockSpec((c, NA * NO), lambda i, j: (0, 0)),
            pl.BlockSpec((1, NA * NO), lambda i, j: (0, 0)),
        ],
        out_specs=pl.BlockSpec((1, NA, tile, NO), lambda i, j: (i, 0, j, 0)),
        out_shape=jax.ShapeDtypeStruct((bs, NA, hw, NO), jnp.float32),
    )(xr, wt, br)
    return out.reshape(bs, NA, ny, nx, NO)


def kernel(x0, x1, x2, W0, b0, W1, b1, W2, b2):
    y0 = _head(x0, W0, b0, tile=512)
    y1 = _head(x1, W1, b1, tile=512)
    y2 = _head(x2, W2, b2, tile=256)
    return (y0, y1, y2)
